# baseline (speedup 1.0000x reference)
import functools

import jax
import jax.numpy as jnp
from jax import lax
from jax.experimental import pallas as pl
from jax.experimental.pallas import tpu as pltpu
from jax.experimental.pallas import tpu_sc as plsc

N = 8192
D_NODE = 1024
D_EDGE = 512
D_OUT = 1024
J = 16

BM = 256
NB = N // BM
WORK = NB + J - 1


_SC_CHUNK = 64


@functools.cache
def _make_row_gather2(n, d):
    info = plsc.get_sparse_core_info()
    nw = info.num_cores * info.num_subcores
    rows_per_w = n // nw
    nchunks = rows_per_w // _SC_CHUNK
    mesh = plsc.VectorSubcoreMesh(core_axis_name="c", subcore_axis_name="s")

    @functools.partial(
        pl.kernel,
        mesh=mesh,
        out_type=[jax.ShapeDtypeStruct((n, d), jnp.float32)] * 2,
        scratch_types=[
            pltpu.VMEM((_SC_CHUNK,), jnp.int32),
            pltpu.VMEM((_SC_CHUNK, d), jnp.float32),
            pltpu.SemaphoreType.DMA,
        ],
    )
    def gather2(t1, i1, t2, i2, o1, o2, idx_v, rows_v, sem):
        wid = lax.axis_index("s") * info.num_cores + lax.axis_index("c")
        base = wid * rows_per_w
        for tbl, idx, out in ((t1, i1, o1), (t2, i2, o2)):
            for c in range(nchunks):
                off = base + c * _SC_CHUNK
                pltpu.sync_copy(idx.at[pl.ds(off, _SC_CHUNK)], idx_v)
                pltpu.async_copy(tbl.at[idx_v], rows_v, sem).wait()
                pltpu.sync_copy(rows_v, out.at[pl.ds(off, _SC_CHUNK)])

    return gather2


def _row_gather2(t1, i1, t2, i2):
    return _make_row_gather2(t1.shape[0], t1.shape[1])(t1, i1, t2, i2)


def _routing(types_sorted):
    t = jnp.arange(J, dtype=jnp.int32)
    first_row = jnp.searchsorted(types_sorted, t, side="left").astype(jnp.int32)
    last_row = jnp.searchsorted(types_sorted, t, side="right").astype(jnp.int32) - 1
    fb = first_row // BM
    lb = last_row // BM
    nonempty = last_row >= first_row
    counts = jnp.where(nonempty, lb - fb + 1, 0).astype(jnp.int32)
    starts = jnp.concatenate(
        [jnp.zeros((1,), jnp.int32), jnp.cumsum(counts)[:-1].astype(jnp.int32)])
    total = starts[-1] + counts[-1]
    s = jnp.arange(WORK, dtype=jnp.int32)
    tid = jnp.searchsorted(starts, s, side="right").astype(jnp.int32) - 1
    bid = fb[tid] + (s - starts[tid])
    valid = (s < total).astype(jnp.int32)
    last_t = types_sorted[N - 1].astype(jnp.int32)
    bid = jnp.where(valid == 1, bid, NB - 1)
    tid = jnp.where(valid == 1, tid, last_t)
    return bid, tid, valid


def _gmm_body(bid_ref, tid_ref, valid_ref, x_ref, w_ref, b_ref, t_ref, o_ref):
    s = pl.program_id(0)
    bid = bid_ref[s]
    prev_bid = bid_ref[jnp.maximum(s - 1, 0)]
    first = jnp.logical_or(s == 0, bid != prev_bid)
    t = tid_ref[s]
    valid = valid_ref[s] != 0
    mask = jnp.logical_and(t_ref[...] == t, valid)
    y = jnp.dot(x_ref[...], w_ref[0], preferred_element_type=jnp.float32)
    y = y + b_ref[0, 0, :][None, :]
    contrib = jnp.where(mask, y, 0.0)
    o_ref[...] = jnp.where(first, 0.0, o_ref[...]) + contrib


def _grouped_linear(xs, W, b, ts_blocks, bid, tid, valid, interpret=False):
    grid_spec = pltpu.PrefetchScalarGridSpec(
        num_scalar_prefetch=3,
        grid=(WORK,),
        in_specs=[
            pl.BlockSpec((BM, D_NODE), lambda s, bids, tids, vs: (bids[s], 0)),
            pl.BlockSpec((1, D_NODE, D_OUT), lambda s, bids, tids, vs: (tids[s], 0, 0)),
            pl.BlockSpec((1, 1, D_OUT), lambda s, bids, tids, vs: (tids[s], 0, 0)),
            pl.BlockSpec((BM, 1), lambda s, bids, tids, vs: (bids[s], 0)),
        ],
        out_specs=pl.BlockSpec((BM, D_OUT), lambda s, bids, tids, vs: (bids[s], 0)),
    )
    return pl.pallas_call(
        _gmm_body,
        grid_spec=grid_spec,
        out_shape=jax.ShapeDtypeStruct((N, D_OUT), jnp.float32),
        interpret=interpret,
    )(bid, tid, valid, xs, W, b.reshape(J, 1, D_OUT), ts_blocks.reshape(N, 1))


def _final_body(t1_ref, t2_ref, e_ref, we_ref, be_ref, wa_ref, wb_ref, wc_ref,
                bo_ref, o_ref):
    acc = jnp.dot(jnp.maximum(t1_ref[...], 0.0), wa_ref[...],
                  preferred_element_type=jnp.float32)
    acc += jnp.dot(jnp.maximum(t2_ref[...], 0.0), wb_ref[...],
                   preferred_element_type=jnp.float32)
    e = jnp.dot(e_ref[...], we_ref[...], preferred_element_type=jnp.float32)
    e = e + be_ref[...]
    acc += jnp.dot(jnp.maximum(e, 0.0), wc_ref[...],
                   preferred_element_type=jnp.float32)
    o_ref[...] = jnp.maximum(acc + bo_ref[...], 0.0)


def _final(tmp1, tmp2, edges, We, be, WoA, WoB, WoC, bo, interpret=False):
    row = lambda i: (i, 0)
    full = lambda i: (0, 0)
    return pl.pallas_call(
        _final_body,
        grid=(NB,),
        in_specs=[
            pl.BlockSpec((BM, D_OUT), row),
            pl.BlockSpec((BM, D_OUT), row),
            pl.BlockSpec((BM, D_EDGE), row),
            pl.BlockSpec((D_EDGE, D_OUT), full),
            pl.BlockSpec((1, D_OUT), full),
            pl.BlockSpec((D_OUT, D_OUT), full),
            pl.BlockSpec((D_OUT, D_OUT), full),
            pl.BlockSpec((D_OUT, D_OUT), full),
            pl.BlockSpec((1, D_OUT), full),
        ],
        out_specs=pl.BlockSpec((BM, D_OUT), row),
        out_shape=jax.ShapeDtypeStruct((N, D_OUT), jnp.float32),
        interpret=interpret,
    )(tmp1, tmp2, edges, We, be.reshape(1, D_OUT),
      WoA, WoB, WoC, bo.reshape(1, D_OUT))


def kernel(nodes_1, nodes_2, edges, node_types_1, node_types_2,
           W1, b1, W2, b2, We, be, Wo, bo):
    p1 = jnp.argsort(node_types_1).astype(jnp.int32)
    p2 = jnp.argsort(node_types_2).astype(jnp.int32)
    t1s = jnp.sort(node_types_1)
    t2s = jnp.sort(node_types_2)
    iota = jnp.arange(N, dtype=jnp.int32)
    inv1 = jnp.zeros((N,), jnp.int32).at[p1].set(iota)
    inv2 = jnp.zeros((N,), jnp.int32).at[p2].set(iota)

    r1 = _routing(t1s)
    r2 = _routing(t2s)

    x1s, x2s = _row_gather2(nodes_1, p1, nodes_2, p2)
    tmp1s = _grouped_linear(x1s, W1, b1, t1s.reshape(NB, 1, BM), *r1)
    tmp2s = _grouped_linear(x2s, W2, b2, t2s.reshape(NB, 1, BM), *r2)
    tmp1, tmp2 = _row_gather2(tmp1s, inv1, tmp2s, inv2)

    WoA = Wo[:D_OUT]
    WoB = Wo[D_OUT:2 * D_OUT]
    WoC = Wo[2 * D_OUT:]
    return _final(tmp1, tmp2, edges, We, be, WoA, WoB, WoC, bo)


# speedup vs baseline: 1.0332x; 1.0332x over previous
import functools

import jax
import jax.numpy as jnp
from jax import lax
from jax.experimental import pallas as pl
from jax.experimental.pallas import tpu as pltpu
from jax.experimental.pallas import tpu_sc as plsc

N = 8192
D_NODE = 1024
D_EDGE = 512
D_OUT = 1024
J = 16

BM = 256
NB = N // BM
WORK = NB + J - 1


_SC_CHUNK = 64


@functools.cache
def _make_row_gather2(n, d):
    info = plsc.get_sparse_core_info()
    nw = info.num_cores * info.num_subcores
    rows_per_w = n // nw
    nchunks = rows_per_w // _SC_CHUNK
    mesh = plsc.VectorSubcoreMesh(core_axis_name="c", subcore_axis_name="s")

    @functools.partial(
        pl.kernel,
        mesh=mesh,
        out_type=[jax.ShapeDtypeStruct((n, d), jnp.float32)] * 2,
        scratch_types=[
            pltpu.VMEM((_SC_CHUNK,), jnp.int32),
            pltpu.VMEM((_SC_CHUNK, d), jnp.float32),
            pltpu.SemaphoreType.DMA,
        ],
    )
    def gather2(t1, i1, t2, i2, o1, o2, idx_v, rows_v, sem):
        wid = lax.axis_index("s") * info.num_cores + lax.axis_index("c")
        base = wid * rows_per_w
        for tbl, idx, out in ((t1, i1, o1), (t2, i2, o2)):
            for c in range(nchunks):
                off = base + c * _SC_CHUNK
                pltpu.sync_copy(idx.at[pl.ds(off, _SC_CHUNK)], idx_v)
                pltpu.async_copy(tbl.at[idx_v], rows_v, sem).wait()
                pltpu.sync_copy(rows_v, out.at[pl.ds(off, _SC_CHUNK)])

    return gather2


def _row_gather2(t1, i1, t2, i2):
    return _make_row_gather2(t1.shape[0], t1.shape[1])(t1, i1, t2, i2)


@functools.cache
def _make_row_scatter2(n, d):
    info = plsc.get_sparse_core_info()
    nw = info.num_cores * info.num_subcores
    rows_per_w = n // nw
    nchunks = rows_per_w // _SC_CHUNK
    mesh = plsc.VectorSubcoreMesh(core_axis_name="c", subcore_axis_name="s")

    @functools.partial(
        pl.kernel,
        mesh=mesh,
        out_type=[jax.ShapeDtypeStruct((n, d), jnp.float32)] * 2,
        scratch_types=[
            pltpu.VMEM((_SC_CHUNK,), jnp.int32),
            pltpu.VMEM((_SC_CHUNK, d), jnp.float32),
            pltpu.SemaphoreType.DMA,
        ],
    )
    def scatter2(t1, i1, t2, i2, o1, o2, idx_v, rows_v, sem):
        wid = lax.axis_index("s") * info.num_cores + lax.axis_index("c")
        base = wid * rows_per_w
        for tbl, idx, out in ((t1, i1, o1), (t2, i2, o2)):
            for c in range(nchunks):
                off = base + c * _SC_CHUNK
                pltpu.sync_copy(idx.at[pl.ds(off, _SC_CHUNK)], idx_v)
                pltpu.sync_copy(tbl.at[pl.ds(off, _SC_CHUNK)], rows_v)
                pltpu.async_copy(rows_v, out.at[idx_v], sem).wait()

    return scatter2


def _row_scatter2(t1, i1, t2, i2):
    return _make_row_scatter2(t1.shape[0], t1.shape[1])(t1, i1, t2, i2)


def _routing(types_sorted):
    ts = types_sorted.reshape(NB, BM)
    tmin = ts[:, 0]
    tmax = ts[:, -1]
    counts = (tmax - tmin + 1).astype(jnp.int32)
    starts = jnp.concatenate(
        [jnp.zeros((1,), jnp.int32), jnp.cumsum(counts)[:-1].astype(jnp.int32)])
    total = starts[-1] + counts[-1]
    s = jnp.arange(WORK, dtype=jnp.int32)
    bid = jnp.searchsorted(starts, s, side="right").astype(jnp.int32) - 1
    tid = tmin[bid] + (s - starts[bid])
    valid = (s < total).astype(jnp.int32)
    bid = jnp.where(valid == 1, bid, NB - 1)
    tid = jnp.where(valid == 1, tid, tmax[NB - 1])
    return bid, tid, valid


def _perm_metadata(types):
    t32 = types.astype(jnp.int32)
    onehot = (t32[:, None] == jnp.arange(J, dtype=jnp.int32)).astype(jnp.int32)
    csum = jnp.cumsum(onehot, axis=0)
    counts = csum[-1]
    starts = jnp.concatenate(
        [jnp.zeros((1,), jnp.int32), jnp.cumsum(counts)[:-1].astype(jnp.int32)])
    rank = starts[t32] + jnp.take_along_axis(csum, t32[:, None], axis=1)[:, 0] - 1
    ends = (starts + counts).astype(jnp.int32)
    row = jnp.arange(N, dtype=jnp.int32)
    types_sorted = (row[:, None] >= ends[None, :]).sum(axis=1).astype(jnp.int32)
    return rank.astype(jnp.int32), types_sorted


def _gmm_body(bid_ref, tid_ref, valid_ref, x_ref, w_ref, b_ref, t_ref, o_ref):
    s = pl.program_id(0)
    bid = bid_ref[s]
    prev_bid = bid_ref[jnp.maximum(s - 1, 0)]
    first = jnp.logical_or(s == 0, bid != prev_bid)
    t = tid_ref[s]
    valid = valid_ref[s] != 0
    mask = jnp.logical_and(t_ref[...] == t, valid)
    y = jnp.dot(x_ref[...], w_ref[0], preferred_element_type=jnp.float32)
    y = y + b_ref[0, 0, :][None, :]
    contrib = jnp.where(mask, y, 0.0)
    o_ref[...] = jnp.where(first, 0.0, o_ref[...]) + contrib


def _grouped_linear(xs, W, b, ts_blocks, bid, tid, valid, interpret=False):
    grid_spec = pltpu.PrefetchScalarGridSpec(
        num_scalar_prefetch=3,
        grid=(WORK,),
        in_specs=[
            pl.BlockSpec((BM, D_NODE), lambda s, bids, tids, vs: (bids[s], 0)),
            pl.BlockSpec((1, D_NODE, D_OUT), lambda s, bids, tids, vs: (tids[s], 0, 0)),
            pl.BlockSpec((1, 1, D_OUT), lambda s, bids, tids, vs: (tids[s], 0, 0)),
            pl.BlockSpec((BM, 1), lambda s, bids, tids, vs: (bids[s], 0)),
        ],
        out_specs=pl.BlockSpec((BM, D_OUT), lambda s, bids, tids, vs: (bids[s], 0)),
    )
    return pl.pallas_call(
        _gmm_body,
        grid_spec=grid_spec,
        out_shape=jax.ShapeDtypeStruct((N, D_OUT), jnp.float32),
        interpret=interpret,
    )(bid, tid, valid, xs, W, b.reshape(J, 1, D_OUT), ts_blocks.reshape(N, 1))


def _final_body(t1_ref, t2_ref, e_ref, we_ref, be_ref, wa_ref, wb_ref, wc_ref,
                bo_ref, o_ref):
    acc = jnp.dot(jnp.maximum(t1_ref[...], 0.0), wa_ref[...],
                  preferred_element_type=jnp.float32)
    acc += jnp.dot(jnp.maximum(t2_ref[...], 0.0), wb_ref[...],
                   preferred_element_type=jnp.float32)
    e = jnp.dot(e_ref[...], we_ref[...], preferred_element_type=jnp.float32)
    e = e + be_ref[...]
    acc += jnp.dot(jnp.maximum(e, 0.0), wc_ref[...],
                   preferred_element_type=jnp.float32)
    o_ref[...] = jnp.maximum(acc + bo_ref[...], 0.0)


def _final(tmp1, tmp2, edges, We, be, WoA, WoB, WoC, bo, interpret=False):
    row = lambda i: (i, 0)
    full = lambda i: (0, 0)
    return pl.pallas_call(
        _final_body,
        grid=(NB,),
        in_specs=[
            pl.BlockSpec((BM, D_OUT), row),
            pl.BlockSpec((BM, D_OUT), row),
            pl.BlockSpec((BM, D_EDGE), row),
            pl.BlockSpec((D_EDGE, D_OUT), full),
            pl.BlockSpec((1, D_OUT), full),
            pl.BlockSpec((D_OUT, D_OUT), full),
            pl.BlockSpec((D_OUT, D_OUT), full),
            pl.BlockSpec((D_OUT, D_OUT), full),
            pl.BlockSpec((1, D_OUT), full),
        ],
        out_specs=pl.BlockSpec((BM, D_OUT), row),
        out_shape=jax.ShapeDtypeStruct((N, D_OUT), jnp.float32),
        interpret=interpret,
    )(tmp1, tmp2, edges, We, be.reshape(1, D_OUT),
      WoA, WoB, WoC, bo.reshape(1, D_OUT))


def kernel(nodes_1, nodes_2, edges, node_types_1, node_types_2,
           W1, b1, W2, b2, We, be, Wo, bo):
    rank1, t1s = _perm_metadata(node_types_1)
    rank2, t2s = _perm_metadata(node_types_2)

    r1 = _routing(t1s)
    r2 = _routing(t2s)

    x1s, x2s = _row_scatter2(nodes_1, rank1, nodes_2, rank2)
    tmp1s = _grouped_linear(x1s, W1, b1, t1s.reshape(NB, 1, BM), *r1)
    tmp2s = _grouped_linear(x2s, W2, b2, t2s.reshape(NB, 1, BM), *r2)
    tmp1, tmp2 = _row_gather2(tmp1s, rank1, tmp2s, rank2)

    WoA = Wo[:D_OUT]
    WoB = Wo[D_OUT:2 * D_OUT]
    WoC = Wo[2 * D_OUT:]
    return _final(tmp1, tmp2, edges, We, be, WoA, WoB, WoC, bo)


# speedup vs baseline: 1.0862x; 1.0513x over previous
import functools

import jax
import jax.numpy as jnp
from jax import lax
from jax.experimental import pallas as pl
from jax.experimental.pallas import tpu as pltpu
from jax.experimental.pallas import tpu_sc as plsc

N = 8192
D_NODE = 1024
D_EDGE = 512
D_OUT = 1024
J = 16

BM = 512
NB = N // BM
WORK = NB + J - 1


_SC_CHUNK = 64


@functools.cache
def _make_row_gather2(n, d):
    info = plsc.get_sparse_core_info()
    nw = info.num_cores * info.num_subcores
    rows_per_w = n // nw
    nchunks = rows_per_w // _SC_CHUNK
    mesh = plsc.VectorSubcoreMesh(core_axis_name="c", subcore_axis_name="s")

    @functools.partial(
        pl.kernel,
        mesh=mesh,
        out_type=[jax.ShapeDtypeStruct((n, d), jnp.float32)] * 2,
        scratch_types=[
            pltpu.VMEM((_SC_CHUNK,), jnp.int32),
            pltpu.VMEM((_SC_CHUNK, d), jnp.float32),
            pltpu.SemaphoreType.DMA,
        ],
    )
    def gather2(t1, i1, t2, i2, o1, o2, idx_v, rows_v, sem):
        wid = lax.axis_index("s") * info.num_cores + lax.axis_index("c")
        base = wid * rows_per_w
        for tbl, idx, out in ((t1, i1, o1), (t2, i2, o2)):
            for c in range(nchunks):
                off = base + c * _SC_CHUNK
                pltpu.sync_copy(idx.at[pl.ds(off, _SC_CHUNK)], idx_v)
                pltpu.async_copy(tbl.at[idx_v], rows_v, sem).wait()
                pltpu.sync_copy(rows_v, out.at[pl.ds(off, _SC_CHUNK)])

    return gather2


def _row_gather2(t1, i1, t2, i2):
    return _make_row_gather2(t1.shape[0], t1.shape[1])(t1, i1, t2, i2)


def _perm_metadata(types):
    t32 = types.astype(jnp.int32)
    order = jnp.argsort(t32, stable=True).astype(jnp.int32)
    types_sorted = jnp.take(t32, order)
    rank = jnp.zeros((N,), jnp.int32).at[order].set(
        jnp.arange(N, dtype=jnp.int32))
    return order, rank, types_sorted


def _routing(types_sorted):
    ts = types_sorted.reshape(NB, BM)
    tmin = ts[:, 0]
    tmax = ts[:, -1]
    counts = (tmax - tmin + 1).astype(jnp.int32)
    starts = jnp.concatenate(
        [jnp.zeros((1,), jnp.int32), jnp.cumsum(counts)[:-1].astype(jnp.int32)])
    total = starts[-1] + counts[-1]
    s = jnp.arange(WORK, dtype=jnp.int32)
    bid = jnp.searchsorted(starts, s, side="right").astype(jnp.int32) - 1
    tid = tmin[bid] + (s - starts[bid])
    valid = (s < total).astype(jnp.int32)
    bid = jnp.where(valid == 1, bid, NB - 1)
    tid = jnp.where(valid == 1, tid, tmax[NB - 1])
    return bid, tid, valid


def _gmm_body(bid_ref, tid_ref, valid_ref, x_ref, w_ref, b_ref, t_ref, o_ref):
    s = pl.program_id(0)
    bid = bid_ref[s]
    prev_bid = bid_ref[jnp.maximum(s - 1, 0)]
    first = jnp.logical_or(s == 0, bid != prev_bid)
    t = tid_ref[s]
    valid = valid_ref[s] != 0
    mask = jnp.logical_and(t_ref[...] == t, valid)
    y = jnp.dot(x_ref[...], w_ref[0], preferred_element_type=jnp.float32)
    y = y + b_ref[0, 0, :][None, :]
    contrib = jnp.where(mask, y, 0.0)
    o_ref[...] = jnp.where(first, 0.0, o_ref[...]) + contrib


def _grouped_linear(xs, W, b, ts, bid, tid, valid, interpret=False):
    grid_spec = pltpu.PrefetchScalarGridSpec(
        num_scalar_prefetch=3,
        grid=(WORK,),
        in_specs=[
            pl.BlockSpec((BM, D_NODE), lambda s, bids, tids, vs: (bids[s], 0)),
            pl.BlockSpec((1, D_NODE, D_OUT), lambda s, bids, tids, vs: (tids[s], 0, 0)),
            pl.BlockSpec((1, 1, D_OUT), lambda s, bids, tids, vs: (tids[s], 0, 0)),
            pl.BlockSpec((BM, 1), lambda s, bids, tids, vs: (bids[s], 0)),
        ],
        out_specs=pl.BlockSpec((BM, D_OUT), lambda s, bids, tids, vs: (bids[s], 0)),
    )
    return pl.pallas_call(
        _gmm_body,
        grid_spec=grid_spec,
        out_shape=jax.ShapeDtypeStruct((N, D_OUT), jnp.float32),
        interpret=interpret,
    )(bid, tid, valid, xs, W, b.reshape(J, 1, D_OUT), ts.reshape(N, 1))


def _final_body(t1_ref, t2_ref, e_ref, we_ref, be_ref, wa_ref, wb_ref, wc_ref,
                bo_ref, o_ref):
    acc = jnp.dot(jnp.maximum(t1_ref[...], 0.0), wa_ref[...],
                  preferred_element_type=jnp.float32)
    acc += jnp.dot(jnp.maximum(t2_ref[...], 0.0), wb_ref[...],
                   preferred_element_type=jnp.float32)
    e = jnp.dot(e_ref[...], we_ref[...], preferred_element_type=jnp.float32)
    e = e + be_ref[...]
    acc += jnp.dot(jnp.maximum(e, 0.0), wc_ref[...],
                   preferred_element_type=jnp.float32)
    o_ref[...] = jnp.maximum(acc + bo_ref[...], 0.0)


def _final(tmp1, tmp2, edges, We, be, WoA, WoB, WoC, bo, interpret=False):
    row = lambda i: (i, 0)
    full = lambda i: (0, 0)
    return pl.pallas_call(
        _final_body,
        grid=(NB,),
        in_specs=[
            pl.BlockSpec((BM, D_OUT), row),
            pl.BlockSpec((BM, D_OUT), row),
            pl.BlockSpec((BM, D_EDGE), row),
            pl.BlockSpec((D_EDGE, D_OUT), full),
            pl.BlockSpec((1, D_OUT), full),
            pl.BlockSpec((D_OUT, D_OUT), full),
            pl.BlockSpec((D_OUT, D_OUT), full),
            pl.BlockSpec((D_OUT, D_OUT), full),
            pl.BlockSpec((1, D_OUT), full),
        ],
        out_specs=pl.BlockSpec((BM, D_OUT), row),
        out_shape=jax.ShapeDtypeStruct((N, D_OUT), jnp.float32),
        interpret=interpret,
    )(tmp1, tmp2, edges, We, be.reshape(1, D_OUT),
      WoA, WoB, WoC, bo.reshape(1, D_OUT))


def kernel(nodes_1, nodes_2, edges, node_types_1, node_types_2,
           W1, b1, W2, b2, We, be, Wo, bo):
    order1, rank1, t1s = _perm_metadata(node_types_1)
    order2, rank2, t2s = _perm_metadata(node_types_2)

    r1 = _routing(t1s)
    r2 = _routing(t2s)

    x1s, x2s = _row_gather2(nodes_1, order1, nodes_2, order2)
    tmp1s = _grouped_linear(x1s, W1, b1, t1s, *r1)
    tmp2s = _grouped_linear(x2s, W2, b2, t2s, *r2)
    tmp1, tmp2 = _row_gather2(tmp1s, rank1, tmp2s, rank2)

    WoA = Wo[:D_OUT]
    WoB = Wo[D_OUT:2 * D_OUT]
    WoC = Wo[2 * D_OUT:]
    return _final(tmp1, tmp2, edges, We, be, WoA, WoB, WoC, bo)


# speedup vs baseline: 1.1076x; 1.0198x over previous
import functools

import jax
import jax.numpy as jnp
from jax import lax
from jax.experimental import pallas as pl
from jax.experimental.pallas import tpu as pltpu
from jax.experimental.pallas import tpu_sc as plsc

N = 8192
D_NODE = 1024
D_EDGE = 512
D_OUT = 1024
J = 16

BM = 512
NB = N // BM
WORK = NB + J - 1


_SC_CHUNK = 64


@functools.cache
def _make_row_gather(n, d):
    info = plsc.get_sparse_core_info()
    nw = info.num_cores * info.num_subcores
    rows_per_w = n // nw
    nchunks = rows_per_w // _SC_CHUNK
    mesh = plsc.VectorSubcoreMesh(core_axis_name="c", subcore_axis_name="s")

    @functools.partial(
        pl.kernel,
        mesh=mesh,
        out_type=jax.ShapeDtypeStruct((n, d), jnp.float32),
        scratch_types=[
            pltpu.VMEM((_SC_CHUNK,), jnp.int32),
            pltpu.VMEM((_SC_CHUNK, d), jnp.float32),
            pltpu.SemaphoreType.DMA,
        ],
    )
    def gather1(tbl, idx, out, idx_v, rows_v, sem):
        wid = lax.axis_index("s") * info.num_cores + lax.axis_index("c")
        base = wid * rows_per_w
        for c in range(nchunks):
            off = base + c * _SC_CHUNK
            pltpu.sync_copy(idx.at[pl.ds(off, _SC_CHUNK)], idx_v)
            pltpu.async_copy(tbl.at[idx_v], rows_v, sem).wait()
            pltpu.sync_copy(rows_v, out.at[pl.ds(off, _SC_CHUNK)])

    return gather1


def _row_gather(tbl, idx):
    return _make_row_gather(tbl.shape[0], tbl.shape[1])(tbl, idx)


def _perm_metadata(types):
    t32 = types.astype(jnp.int32)
    order = jnp.argsort(t32, stable=True).astype(jnp.int32)
    types_sorted = jnp.take(t32, order)
    rank = jnp.zeros((N,), jnp.int32).at[order].set(
        jnp.arange(N, dtype=jnp.int32))
    return order, rank, types_sorted


def _routing(types_sorted):
    ts = types_sorted.reshape(NB, BM)
    tmin = ts[:, 0]
    tmax = ts[:, -1]
    counts = (tmax - tmin + 1).astype(jnp.int32)
    starts = jnp.concatenate(
        [jnp.zeros((1,), jnp.int32), jnp.cumsum(counts)[:-1].astype(jnp.int32)])
    total = starts[-1] + counts[-1]
    s = jnp.arange(WORK, dtype=jnp.int32)
    bid = jnp.searchsorted(starts, s, side="right").astype(jnp.int32) - 1
    tid = tmin[bid] + (s - starts[bid])
    valid = (s < total).astype(jnp.int32)
    bid = jnp.where(valid == 1, bid, NB - 1)
    tid = jnp.where(valid == 1, tid, tmax[NB - 1])
    return bid, tid, valid


def _gmm_body(bid_ref, tid_ref, valid_ref, x_ref, w_ref, b_ref, t_ref, o_ref):
    s = pl.program_id(0)
    bid = bid_ref[s]
    prev_bid = bid_ref[jnp.maximum(s - 1, 0)]
    first = jnp.logical_or(s == 0, bid != prev_bid)
    t = tid_ref[s]
    valid = valid_ref[s] != 0
    mask = jnp.logical_and(t_ref[...] == t, valid)
    y = jnp.dot(x_ref[...], w_ref[0], preferred_element_type=jnp.float32)
    y = y + b_ref[0, 0, :][None, :]
    contrib = jnp.where(mask, y, 0.0)
    o_ref[...] = jnp.where(first, 0.0, o_ref[...]) + contrib


def _grouped_linear(xs, W, b, ts, bid, tid, valid, interpret=False):
    grid_spec = pltpu.PrefetchScalarGridSpec(
        num_scalar_prefetch=3,
        grid=(WORK,),
        in_specs=[
            pl.BlockSpec((BM, D_NODE), lambda s, bids, tids, vs: (bids[s], 0)),
            pl.BlockSpec((1, D_NODE, D_OUT), lambda s, bids, tids, vs: (tids[s], 0, 0)),
            pl.BlockSpec((1, 1, D_OUT), lambda s, bids, tids, vs: (tids[s], 0, 0)),
            pl.BlockSpec((BM, 1), lambda s, bids, tids, vs: (bids[s], 0)),
        ],
        out_specs=pl.BlockSpec((BM, D_OUT), lambda s, bids, tids, vs: (bids[s], 0)),
    )
    return pl.pallas_call(
        _gmm_body,
        grid_spec=grid_spec,
        out_shape=jax.ShapeDtypeStruct((N, D_OUT), jnp.float32),
        interpret=interpret,
    )(bid, tid, valid, xs, W, b.reshape(J, 1, D_OUT), ts.reshape(N, 1))


def _edge_part_body(e_ref, we_ref, be_ref, wc_ref, bo_ref, o_ref):
    e = jnp.dot(e_ref[...], we_ref[...], preferred_element_type=jnp.float32)
    e = e + be_ref[...]
    acc = jnp.dot(jnp.maximum(e, 0.0), wc_ref[...],
                  preferred_element_type=jnp.float32)
    o_ref[...] = acc + bo_ref[...]


def _edge_part(edges, We, be, WoC, bo, interpret=False):
    row = lambda i: (i, 0)
    full = lambda i: (0, 0)
    return pl.pallas_call(
        _edge_part_body,
        grid=(NB,),
        in_specs=[
            pl.BlockSpec((BM, D_EDGE), row),
            pl.BlockSpec((D_EDGE, D_OUT), full),
            pl.BlockSpec((1, D_OUT), full),
            pl.BlockSpec((D_OUT, D_OUT), full),
            pl.BlockSpec((1, D_OUT), full),
        ],
        out_specs=pl.BlockSpec((BM, D_OUT), row),
        out_shape=jax.ShapeDtypeStruct((N, D_OUT), jnp.float32),
        interpret=interpret,
    )(edges, We, be.reshape(1, D_OUT), WoC, bo.reshape(1, D_OUT))


def _final_body(t1_ref, t2_ref, eacc_ref, wa_ref, wb_ref, o_ref):
    acc = jnp.dot(jnp.maximum(t1_ref[...], 0.0), wa_ref[...],
                  preferred_element_type=jnp.float32)
    acc += jnp.dot(jnp.maximum(t2_ref[...], 0.0), wb_ref[...],
                   preferred_element_type=jnp.float32)
    o_ref[...] = jnp.maximum(acc + eacc_ref[...], 0.0)


def _final(tmp1, tmp2, eacc, WoA, WoB, interpret=False):
    row = lambda i: (i, 0)
    full = lambda i: (0, 0)
    return pl.pallas_call(
        _final_body,
        grid=(NB,),
        in_specs=[
            pl.BlockSpec((BM, D_OUT), row),
            pl.BlockSpec((BM, D_OUT), row),
            pl.BlockSpec((BM, D_OUT), row),
            pl.BlockSpec((D_OUT, D_OUT), full),
            pl.BlockSpec((D_OUT, D_OUT), full),
        ],
        out_specs=pl.BlockSpec((BM, D_OUT), row),
        out_shape=jax.ShapeDtypeStruct((N, D_OUT), jnp.float32),
        interpret=interpret,
    )(tmp1, tmp2, eacc, WoA, WoB)


def kernel(nodes_1, nodes_2, edges, node_types_1, node_types_2,
           W1, b1, W2, b2, We, be, Wo, bo):
    order1, rank1, t1s = _perm_metadata(node_types_1)
    order2, rank2, t2s = _perm_metadata(node_types_2)

    r1 = _routing(t1s)
    r2 = _routing(t2s)

    WoA = Wo[:D_OUT]
    WoB = Wo[D_OUT:2 * D_OUT]
    WoC = Wo[2 * D_OUT:]

    x1s = _row_gather(nodes_1, order1)
    eacc = _edge_part(edges, We, be, WoC, bo)
    x2s = _row_gather(nodes_2, order2)
    tmp1s = _grouped_linear(x1s, W1, b1, t1s, *r1)
    tmp1 = _row_gather(tmp1s, rank1)
    tmp2s = _grouped_linear(x2s, W2, b2, t2s, *r2)
    tmp2 = _row_gather(tmp2s, rank2)
    return _final(tmp1, tmp2, eacc, WoA, WoB)
